# Initial kernel scaffold; baseline (speedup 1.0000x reference)
import jax
import jax.numpy as jnp
from jax.experimental import pallas as pl


def kernel(x, edge_index, w, bias):
    raise NotImplementedError("write your pallas kernel here")


# speedup vs baseline: 3.3370x; 3.3370x over previous
import functools

import jax
import jax.numpy as jnp
from jax import lax
from jax.experimental import pallas as pl
from jax.experimental.pallas import tpu as pltpu
from jax.experimental.pallas import tpu_sc as plsc

D = 128
LANES = 16
NC, NS = 2, 16
NW = NC * NS
CHUNK = 128
ZROWS = 128


def _sc_partial(x, srcp, dstp, wp, chunks_per_worker, n_acc):
    rows_per_tile = n_acc // NS
    mesh = plsc.VectorSubcoreMesh(core_axis_name="c", subcore_axis_name="s")

    @functools.partial(
        pl.kernel,
        out_type=jax.ShapeDtypeStruct((NC, n_acc, D), jnp.float32),
        mesh=mesh,
        scratch_types=[
            pltpu.VMEM((CHUNK,), jnp.int32),
            pltpu.VMEM((CHUNK,), jnp.int32),
            pltpu.VMEM((CHUNK,), jnp.float32),
            pltpu.VMEM((CHUNK, D), jnp.float32),
            pltpu.VMEM((ZROWS, D), jnp.float32),
            pltpu.VMEM_SHARED((n_acc, D), jnp.float32),
            pltpu.SemaphoreType.DMA,
        ],
        compiler_params=pltpu.CompilerParams(needs_layout_passes=False),
    )
    def k(x_hbm, src_hbm, dst_hbm, w_hbm, part_hbm,
          sidx_v, didx_v, w_v, rows_v, zero_v, acc_sh, sem):
        c = lax.axis_index("c")
        s = lax.axis_index("s")
        wid = c * NS + s

        def zfill(r, _):
            for j in range(D // LANES):
                zero_v[r, pl.ds(j * LANES, LANES)] = jnp.zeros((LANES,), jnp.float32)
            return 0
        lax.fori_loop(0, ZROWS, zfill, 0)

        def zcopy(kk, _):
            pltpu.sync_copy(
                zero_v, acc_sh.at[pl.ds(s * rows_per_tile + kk * ZROWS, ZROWS)])
            return 0
        lax.fori_loop(0, rows_per_tile // ZROWS, zcopy, 0)
        plsc.subcore_barrier()

        def chunk_body(i, _):
            base = (wid * chunks_per_worker + i) * CHUNK
            pltpu.sync_copy(src_hbm.at[pl.ds(base, CHUNK)], sidx_v)
            pltpu.sync_copy(dst_hbm.at[pl.ds(base, CHUNK)], didx_v)
            pltpu.sync_copy(w_hbm.at[pl.ds(base, CHUNK)], w_v)
            pltpu.async_copy(x_hbm.at[sidx_v], rows_v, sem).wait()

            def row_body(e, _):
                wsplat = plsc.load_gather(
                    w_v, [jnp.full((LANES,), e, jnp.int32)])
                for j in range(D // LANES):
                    sl = pl.ds(j * LANES, LANES)
                    rows_v[e, sl] = rows_v[e, sl] * wsplat
                return 0
            lax.fori_loop(0, CHUNK, row_body, 0)

            pltpu.sync_copy(rows_v, acc_sh.at[didx_v], add=True)
            return 0
        lax.fori_loop(0, chunks_per_worker, chunk_body, 0)
        plsc.subcore_barrier()

        pltpu.sync_copy(
            acc_sh.at[pl.ds(s * rows_per_tile, rows_per_tile)],
            part_hbm.at[c, pl.ds(s * rows_per_tile, rows_per_tile)])

    return k(x, srcp, dstp, wp)


def _finish_body(p0_ref, p1_ref, b_ref, o_ref):
    o_ref[...] = jnp.maximum(p0_ref[...] + p1_ref[...] + b_ref[...], 0.0)


def kernel(x, edge_index, w, bias):
    n_nodes = x.shape[0]
    e = w.shape[0]
    n_chunks = -(-e // CHUNK)
    chunks_per_worker = -(-n_chunks // NW)
    e_pad = chunks_per_worker * NW * CHUNK
    pad = e_pad - e

    src = jnp.pad(edge_index[0], (0, pad))
    dst = jnp.pad(edge_index[1], (0, pad))
    wp = jnp.pad(w, (0, pad))

    n_acc = -(-n_nodes // (8 * NS)) * (8 * NS)
    partial = _sc_partial(x, src, dst, wp, chunks_per_worker, n_acc)

    blk = 1000
    grid = n_nodes // blk
    out = pl.pallas_call(
        _finish_body,
        grid=(grid,),
        in_specs=[
            pl.BlockSpec((blk, D), lambda i: (i, 0)),
            pl.BlockSpec((blk, D), lambda i: (i, 0)),
            pl.BlockSpec((blk, 1), lambda i: (i, 0)),
        ],
        out_specs=pl.BlockSpec((blk, D), lambda i: (i, 0)),
        out_shape=jax.ShapeDtypeStruct((n_nodes, D), jnp.float32),
    )(partial[0, :n_nodes], partial[1, :n_nodes], bias[:, None])
    return out
